# baseline (speedup 1.0000x reference)
import functools

import jax
import jax.numpy as jnp
from jax import lax
from jax.experimental import pallas as pl
from jax.experimental.pallas import tpu as pltpu
from jax.experimental.pallas import tpu_sc as plsc

_LANES = 16


@functools.cache
def _make_sc_add(total, pe_total, n_workers, n_cores, chunk):
    elems_w = total // n_workers
    n_chunks = elems_w // chunk
    workers_per_batch = pe_total // elems_w

    mesh = plsc.VectorSubcoreMesh(core_axis_name="c", subcore_axis_name="s")

    @functools.partial(
        pl.kernel,
        out_type=jax.ShapeDtypeStruct((total,), jnp.float32),
        mesh=mesh,
        scratch_types=(
            [pltpu.VMEM((chunk,), jnp.float32) for _ in range(6)]
            + [pltpu.SemaphoreType.DMA for _ in range(6)]
        ),
    )
    def sc_add(x_hbm, pe_hbm, o_hbm,
               xb0, xb1, pb0, pb1, ob0, ob1,
               six0, six1, sip0, sip1, so0, so1):
        w = lax.axis_index("s") * n_cores + lax.axis_index("c")
        xbase = w * elems_w
        pbase = (w % workers_per_batch) * elems_w
        xbufs = (xb0, xb1)
        pbufs = (pb0, pb1)
        obufs = (ob0, ob1)
        sin_x = (six0, six1)
        sin_p = (sip0, sip1)
        souts = (so0, so1)

        def in_copies(c, b):
            off = c * chunk
            cx = pltpu.make_async_copy(
                x_hbm.at[pl.ds(xbase + off, chunk)], xbufs[b], sin_x[b])
            cp = pltpu.make_async_copy(
                pe_hbm.at[pl.ds(pbase + off, chunk)], pbufs[b], sin_p[b])
            return cx, cp

        def out_copy(c, b):
            return pltpu.make_async_copy(
                obufs[b], o_hbm.at[pl.ds(xbase + c * chunk, chunk)], souts[b])

        for b in range(2):
            cx, cp = in_copies(b, b)
            cx.start()
            cp.start()

        @pl.loop(0, n_chunks, step=2)
        def _chunk_loop(cc):
            for b in range(2):
                c = cc + b
                cx, cp = in_copies(c, b)
                cx.wait()
                cp.wait()
                @pl.when(c >= 2)
                def _():
                    out_copy(c - 2, b).wait()

                xb, pb, ob = xbufs[b], pbufs[b], obufs[b]

                @plsc.parallel_loop(0, chunk, step=_LANES, unroll=8)
                def _(i):
                    ob[pl.ds(i, _LANES)] = (
                        xb[pl.ds(i, _LANES)] + pb[pl.ds(i, _LANES)])

                out_copy(c, b).start()

                @pl.when(c + 2 < n_chunks)
                def _():
                    ncx, ncp = in_copies(c + 2, b)
                    ncx.start()
                    ncp.start()

        for b in range(2):
            out_copy(n_chunks - 2 + b, b).wait()

    return sc_add


def kernel(x, pe):
    batch, seq_len, d_model = x.shape
    total = batch * seq_len * d_model
    pe_total = seq_len * d_model
    n_workers = 32
    n_cores = 2
    chunk = 16384

    x_flat = x.reshape(total)
    pe_flat = pe[:seq_len].reshape(pe_total)
    fn = _make_sc_add(total, pe_total, n_workers, n_cores, chunk)
    out = fn(x_flat, pe_flat)
    return out.reshape(x.shape)


# speedup vs baseline: 2.6121x; 2.6121x over previous
import functools

import jax
import jax.numpy as jnp
from jax import lax
from jax.experimental import pallas as pl
from jax.experimental.pallas import tpu as pltpu
from jax.experimental.pallas import tpu_sc as plsc

_LANES = 16


@functools.cache
def _make_sc_add(n_rows, pe_rows, d_model, n_workers, n_cores, chunk_rows):
    rows_w = n_rows // n_workers
    n_chunks = rows_w // chunk_rows
    workers_per_batch = pe_rows // rows_w
    groups = chunk_rows * d_model // _LANES
    gpr = d_model // _LANES

    mesh = plsc.VectorSubcoreMesh(core_axis_name="c", subcore_axis_name="s")

    @functools.partial(
        pl.kernel,
        out_type=jax.ShapeDtypeStruct((n_rows, d_model), jnp.float32),
        mesh=mesh,
        scratch_types=(
            [pltpu.VMEM((chunk_rows, d_model), jnp.float32) for _ in range(6)]
            + [pltpu.SemaphoreType.DMA for _ in range(6)]
        ),
        compiler_params=pltpu.CompilerParams(use_tc_tiling_on_sc=True),
    )
    def sc_add(x_hbm, pe_hbm, o_hbm,
               xb0, xb1, pb0, pb1, ob0, ob1,
               six0, six1, sip0, sip1, so0, so1):
        w = lax.axis_index("s") * n_cores + lax.axis_index("c")
        xbase = w * rows_w
        pbase = (w % workers_per_batch) * rows_w
        xbufs = (xb0, xb1)
        pbufs = (pb0, pb1)
        obufs = (ob0, ob1)
        sin_x = (six0, six1)
        sin_p = (sip0, sip1)
        souts = (so0, so1)

        def in_copies(c, b):
            r = c * chunk_rows
            cx = pltpu.make_async_copy(
                x_hbm.at[pl.ds(xbase + r, chunk_rows)], xbufs[b], sin_x[b])
            cp = pltpu.make_async_copy(
                pe_hbm.at[pl.ds(pbase + r, chunk_rows)], pbufs[b], sin_p[b])
            return cx, cp

        def out_copy(c, b):
            return pltpu.make_async_copy(
                obufs[b], o_hbm.at[pl.ds(xbase + c * chunk_rows, chunk_rows)],
                souts[b])

        for b in range(2):
            cx, cp = in_copies(b, b)
            cx.start()
            cp.start()

        @pl.loop(0, n_chunks, step=2)
        def _chunk_loop(cc):
            for b in range(2):
                c = cc + b
                cx, cp = in_copies(c, b)
                cx.wait()
                cp.wait()
                @pl.when(c >= 2)
                def _():
                    out_copy(c - 2, b).wait()

                xb, pb, ob = xbufs[b], pbufs[b], obufs[b]

                @plsc.parallel_loop(0, groups, step=1, unroll=8)
                def _(g):
                    r = g // gpr
                    j = (g % gpr) * _LANES
                    ob[r, pl.ds(j, _LANES)] = (
                        xb[r, pl.ds(j, _LANES)] + pb[r, pl.ds(j, _LANES)])

                out_copy(c, b).start()

                @pl.when(c + 2 < n_chunks)
                def _():
                    ncx, ncp = in_copies(c + 2, b)
                    ncx.start()
                    ncp.start()

        for b in range(2):
            out_copy(n_chunks - 2 + b, b).wait()

    return sc_add


def kernel(x, pe):
    batch, seq_len, d_model = x.shape
    n_rows = batch * seq_len
    n_workers = 32
    n_cores = 2
    chunk_rows = 16

    x2 = x.reshape(n_rows, d_model)
    pe2 = pe[:seq_len]
    fn = _make_sc_add(n_rows, seq_len, d_model, n_workers, n_cores,
                      chunk_rows)
    out = fn(x2, pe2)
    return out.reshape(x.shape)


# speedup vs baseline: 3.2192x; 1.2324x over previous
import functools

import jax
import jax.numpy as jnp
from jax import lax
from jax.experimental import pallas as pl
from jax.experimental.pallas import tpu as pltpu
from jax.experimental.pallas import tpu_sc as plsc

_LANES = 16


@functools.cache
def _make_sc_add(batch, seq_len, d_model, n_workers, n_cores, chunk_rows):
    n_rows = batch * seq_len
    s_w = seq_len // n_workers
    n_sc = s_w // chunk_rows
    n_steps = n_sc * batch
    groups = chunk_rows * d_model // _LANES
    gpr = d_model // _LANES

    mesh = plsc.VectorSubcoreMesh(core_axis_name="c", subcore_axis_name="s")

    @functools.partial(
        pl.kernel,
        out_type=jax.ShapeDtypeStruct((n_rows, d_model), jnp.float32),
        mesh=mesh,
        scratch_types=(
            [pltpu.VMEM((chunk_rows, d_model), jnp.float32) for _ in range(6)]
            + [pltpu.SemaphoreType.DMA for _ in range(6)]
        ),
        compiler_params=pltpu.CompilerParams(use_tc_tiling_on_sc=True),
    )
    def sc_add(x_hbm, pe_hbm, o_hbm,
               xb0, xb1, pb0, pb1, ob0, ob1,
               six0, six1, sip0, sip1, so0, so1):
        w = lax.axis_index("s") * n_cores + lax.axis_index("c")
        sbase = w * s_w
        xbufs = (xb0, xb1)
        pbufs = (pb0, pb1)
        obufs = (ob0, ob1)
        sin_x = (six0, six1)
        sin_p = (sip0, sip1)
        souts = (so0, so1)

        def xrow(t):
            return (t % batch) * seq_len + sbase + (t // batch) * chunk_rows

        def x_copy(t, b):
            return pltpu.make_async_copy(
                x_hbm.at[pl.ds(xrow(t), chunk_rows)], xbufs[b], sin_x[b])

        def pe_copy(sc, b):
            return pltpu.make_async_copy(
                pe_hbm.at[pl.ds(sbase + sc * chunk_rows, chunk_rows)],
                pbufs[b], sin_p[b])

        def out_copy(t, b):
            return pltpu.make_async_copy(
                obufs[b], o_hbm.at[pl.ds(xrow(t), chunk_rows)], souts[b])

        for b in range(2):
            x_copy(b, b).start()
            pe_copy(b, b).start()

        @pl.loop(0, n_steps, step=2 * batch)
        def _step_loop(tt):
            for q in range(2 * batch):
                t = tt + q
                b = q % 2
                pset = (q // batch) % 2
                sc = t // batch

                if q % batch == 0:
                    pe_copy(sc, pset).wait()

                x_copy(t, b).wait()
                @pl.when(t >= 2)
                def _():
                    out_copy(t - 2, b).wait()

                xb, pb, ob = xbufs[b], pbufs[pset], obufs[b]

                @plsc.parallel_loop(0, groups, step=1, unroll=8)
                def _(g):
                    r = g // gpr
                    j = (g % gpr) * _LANES
                    ob[r, pl.ds(j, _LANES)] = (
                        xb[r, pl.ds(j, _LANES)] + pb[r, pl.ds(j, _LANES)])

                out_copy(t, b).start()

                @pl.when(t + 2 < n_steps)
                def _():
                    x_copy(t + 2, b).start()

                if q % batch == batch - 1:
                    @pl.when(sc + 2 < n_sc)
                    def _():
                        pe_copy(sc + 2, pset).start()

        for b in range(2):
            out_copy(n_steps - 2 + b, b).wait()

    return sc_add


def kernel(x, pe):
    batch, seq_len, d_model = x.shape
    n_workers = 32
    n_cores = 2
    chunk_rows = 16

    x2 = x.reshape(batch * seq_len, d_model)
    pe2 = pe[:seq_len]
    fn = _make_sc_add(batch, seq_len, d_model, n_workers, n_cores,
                      chunk_rows)
    out = fn(x2, pe2)
    return out.reshape(x.shape)
